# baseline (speedup 1.0000x reference)
import jax
import jax.numpy as jnp
from jax.experimental import pallas as pl
from jax.experimental.pallas import tpu as pltpu

_TM = 200
_NBUF = 4


def _gcn_body(feat_hbm, w_hbm, adj_hbm, bias_hbm, out_hbm,
              bufs, featv, wv, biasv, fwv, outv,
              adj_sems, aux_sem, out_sems):
    n = adj_hbm.shape[0]
    tm = bufs.shape[1]
    nbuf = bufs.shape[0]
    nblk = n // tm

    cp_feat = pltpu.make_async_copy(feat_hbm, featv, aux_sem)
    cp_feat.start()
    cp_w = pltpu.make_async_copy(w_hbm, wv, aux_sem)
    cp_w.start()
    pltpu.make_async_copy(adj_hbm.at[pl.ds(0, tm), :],
                          bufs.at[0], adj_sems.at[0]).start()
    cp_bias = pltpu.make_async_copy(bias_hbm, biasv, aux_sem)
    cp_bias.start()
    for b in range(1, min(nbuf, nblk)):
        pltpu.make_async_copy(adj_hbm.at[pl.ds(b * tm, tm), :],
                              bufs.at[b], adj_sems.at[b]).start()
    cp_feat.wait()
    cp_w.wait()

    fwv[...] = jnp.dot(featv[...], wv[...], preferred_element_type=jnp.float32)
    cp_bias.wait()

    def step(i, carry):
        slot = jax.lax.rem(i, nbuf)
        pltpu.make_async_copy(adj_hbm.at[pl.ds(i * tm, tm), :],
                              bufs.at[slot], adj_sems.at[slot]).wait()
        acc = jnp.dot(bufs[slot], fwv[...], preferred_element_type=jnp.float32)

        @pl.when(i >= nbuf)
        def _():
            pltpu.make_async_copy(outv.at[slot],
                                  out_hbm.at[pl.ds((i - nbuf) * tm, tm), :],
                                  out_sems.at[slot]).wait()

        outv[slot] = jnp.maximum(acc + biasv[pl.ds(i * tm, tm), :], 0.0)
        pltpu.make_async_copy(outv.at[slot],
                              out_hbm.at[pl.ds(i * tm, tm), :],
                              out_sems.at[slot]).start()

        @pl.when(i + nbuf < nblk)
        def _():
            pltpu.make_async_copy(adj_hbm.at[pl.ds((i + nbuf) * tm, tm), :],
                                  bufs.at[slot], adj_sems.at[slot]).start()
        return carry

    jax.lax.fori_loop(0, nblk, step, 0)

    for b in range(min(nbuf, nblk)):
        i = nblk - min(nbuf, nblk) + b
        slot = i % nbuf
        pltpu.make_async_copy(outv.at[slot],
                              out_hbm.at[pl.ds(i * tm, tm), :],
                              out_sems.at[slot]).wait()


def kernel(adj, feature, weight, bias):
    n, d = feature.shape
    f = weight.shape[1]
    tm, nbuf = _TM, _NBUF
    hbm = pl.BlockSpec(memory_space=pltpu.MemorySpace.HBM)
    return pl.pallas_call(
        _gcn_body,
        in_specs=[hbm, hbm, hbm, hbm],
        out_specs=pl.BlockSpec(memory_space=pltpu.MemorySpace.HBM),
        out_shape=jax.ShapeDtypeStruct((n, f), jnp.float32),
        scratch_shapes=[
            pltpu.VMEM((nbuf, tm, n), jnp.float32),
            pltpu.VMEM((n, d), jnp.float32),
            pltpu.VMEM((d, f), jnp.float32),
            pltpu.VMEM((n, f), jnp.float32),
            pltpu.VMEM((n, f), jnp.float32),
            pltpu.VMEM((nbuf, tm, f), jnp.float32),
            pltpu.SemaphoreType.DMA((nbuf,)),
            pltpu.SemaphoreType.DMA,
            pltpu.SemaphoreType.DMA((nbuf,)),
        ],
    )(feature, weight, adj, bias)


# speedup vs baseline: 1.0175x; 1.0175x over previous
import jax
import jax.numpy as jnp
from jax.experimental import pallas as pl
from jax.experimental.pallas import tpu as pltpu

_TM = 200
_NBUF = 3


def _gcn_body(feat_hbm, w_hbm, adj_hbm, bias_hbm, out_hbm,
              bufs, featv, wv, biasv, fwv, outv,
              adj_sems, aux_sem, out_sems):
    n = adj_hbm.shape[0]
    tm = bufs.shape[1]
    nbuf = bufs.shape[0]
    nblk = n // tm

    cp_feat = pltpu.make_async_copy(feat_hbm, featv, aux_sem)
    cp_feat.start()
    cp_w = pltpu.make_async_copy(w_hbm, wv, aux_sem)
    cp_w.start()
    pltpu.make_async_copy(adj_hbm.at[pl.ds(0, tm), :],
                          bufs.at[0], adj_sems.at[0]).start()
    cp_bias = pltpu.make_async_copy(bias_hbm, biasv, aux_sem)
    cp_bias.start()
    for b in range(1, min(nbuf, nblk)):
        pltpu.make_async_copy(adj_hbm.at[pl.ds(b * tm, tm), :],
                              bufs.at[b], adj_sems.at[b]).start()
    cp_feat.wait()
    cp_w.wait()

    fwv[...] = jnp.dot(featv[...], wv[...], preferred_element_type=jnp.float32)
    cp_bias.wait()

    def step(i, carry):
        slot = jax.lax.rem(i, nbuf)
        pltpu.make_async_copy(adj_hbm.at[pl.ds(i * tm, tm), :],
                              bufs.at[slot], adj_sems.at[slot]).wait()
        acc = jnp.dot(bufs[slot], fwv[...], preferred_element_type=jnp.float32)

        @pl.when(i >= nbuf)
        def _():
            pltpu.make_async_copy(outv.at[slot],
                                  out_hbm.at[pl.ds((i - nbuf) * tm, tm), :],
                                  out_sems.at[slot]).wait()

        outv[slot] = jnp.maximum(acc + biasv[pl.ds(i * tm, tm), :], 0.0)
        pltpu.make_async_copy(outv.at[slot],
                              out_hbm.at[pl.ds(i * tm, tm), :],
                              out_sems.at[slot]).start()

        @pl.when(i + nbuf < nblk)
        def _():
            pltpu.make_async_copy(adj_hbm.at[pl.ds((i + nbuf) * tm, tm), :],
                                  bufs.at[slot], adj_sems.at[slot]).start()
        return carry

    jax.lax.fori_loop(0, nblk, step, 0)

    for b in range(min(nbuf, nblk)):
        i = nblk - min(nbuf, nblk) + b
        slot = i % nbuf
        pltpu.make_async_copy(outv.at[slot],
                              out_hbm.at[pl.ds(i * tm, tm), :],
                              out_sems.at[slot]).wait()


def kernel(adj, feature, weight, bias):
    n, d = feature.shape
    f = weight.shape[1]
    tm, nbuf = _TM, _NBUF
    hbm = pl.BlockSpec(memory_space=pltpu.MemorySpace.HBM)
    return pl.pallas_call(
        _gcn_body,
        in_specs=[hbm, hbm, hbm, hbm],
        out_specs=pl.BlockSpec(memory_space=pltpu.MemorySpace.HBM),
        out_shape=jax.ShapeDtypeStruct((n, f), jnp.float32),
        scratch_shapes=[
            pltpu.VMEM((nbuf, tm, n), jnp.float32),
            pltpu.VMEM((n, d), jnp.float32),
            pltpu.VMEM((d, f), jnp.float32),
            pltpu.VMEM((n, f), jnp.float32),
            pltpu.VMEM((n, f), jnp.float32),
            pltpu.VMEM((nbuf, tm, f), jnp.float32),
            pltpu.SemaphoreType.DMA((nbuf,)),
            pltpu.SemaphoreType.DMA,
            pltpu.SemaphoreType.DMA((nbuf,)),
        ],
    )(feature, weight, adj, bias)


# speedup vs baseline: 1.0273x; 1.0096x over previous
import jax
import jax.numpy as jnp
from jax.experimental import pallas as pl
from jax.experimental.pallas import tpu as pltpu

_TM = 200
_NBUF = 3


def _gcn_body(feat_hbm, w_hbm, adj_hbm, bias_hbm, out_hbm,
              bufs, featv, wv, biasv, fwv, outv,
              adj_sems, aux_sem, out_sems):
    n = adj_hbm.shape[0]
    tm = bufs.shape[1]
    nbuf = bufs.shape[0]
    nblk = n // tm
    th0 = (tm // 2 + 7) // 8 * 8
    th1 = tm - th0

    cp_feat = pltpu.make_async_copy(feat_hbm, featv, aux_sem)
    cp_feat.start()
    cp_w = pltpu.make_async_copy(w_hbm, wv, aux_sem)
    cp_w.start()
    pltpu.make_async_copy(adj_hbm.at[pl.ds(0, tm), :],
                          bufs.at[0], adj_sems.at[0]).start()
    cp_bias = pltpu.make_async_copy(bias_hbm, biasv, aux_sem)
    cp_bias.start()
    for b in range(1, min(nbuf, nblk)):
        pltpu.make_async_copy(adj_hbm.at[pl.ds(b * tm, tm), :],
                              bufs.at[b], adj_sems.at[b]).start()
    cp_feat.wait()
    cp_w.wait()

    fwv[...] = jnp.dot(featv[...], wv[...], preferred_element_type=jnp.float32)
    cp_bias.wait()

    def step(i, carry):
        slot = jax.lax.rem(i, nbuf)
        pltpu.make_async_copy(adj_hbm.at[pl.ds(i * tm, tm), :],
                              bufs.at[slot], adj_sems.at[slot]).wait()
        acc = jnp.dot(bufs[slot], fwv[...], preferred_element_type=jnp.float32)

        @pl.when(i >= nbuf)
        def _():
            pltpu.make_async_copy(outv.at[slot],
                                  out_hbm.at[pl.ds((i - nbuf) * tm, tm), :],
                                  out_sems.at[slot]).wait()

        outv[slot] = jnp.maximum(acc + biasv[pl.ds(i * tm, tm), :], 0.0)
        pltpu.make_async_copy(outv.at[slot],
                              out_hbm.at[pl.ds(i * tm, tm), :],
                              out_sems.at[slot]).start()

        @pl.when(i + nbuf < nblk - 1)
        def _():
            pltpu.make_async_copy(adj_hbm.at[pl.ds((i + nbuf) * tm, tm), :],
                                  bufs.at[slot], adj_sems.at[slot]).start()

        @pl.when(i + nbuf == nblk - 1)
        def _():
            base = (nblk - 1) * tm
            pltpu.make_async_copy(adj_hbm.at[pl.ds(base, th0), :],
                                  bufs.at[slot, pl.ds(0, th0), :],
                                  adj_sems.at[slot]).start()
            pltpu.make_async_copy(adj_hbm.at[pl.ds(base + th0, th1), :],
                                  bufs.at[slot, pl.ds(th0, th1), :],
                                  adj_sems.at[slot]).start()
        return carry

    jax.lax.fori_loop(0, nblk - 1, step, 0)

    last = nblk - 1
    lslot = last % nbuf
    lbase = last * tm
    if last >= nbuf:
        pltpu.make_async_copy(outv.at[lslot],
                              out_hbm.at[pl.ds((last - nbuf) * tm, tm), :],
                              out_sems.at[lslot]).wait()
    for off, sz in ((0, th0), (th0, th1)):
        pltpu.make_async_copy(adj_hbm.at[pl.ds(lbase + off, sz), :],
                              bufs.at[lslot, pl.ds(off, sz), :],
                              adj_sems.at[lslot]).wait()
        acc = jnp.dot(bufs[lslot, pl.ds(off, sz), :], fwv[...],
                      preferred_element_type=jnp.float32)
        outv[lslot, pl.ds(off, sz), :] = jnp.maximum(
            acc + biasv[pl.ds(lbase + off, sz), :], 0.0)
    pltpu.make_async_copy(outv.at[lslot],
                          out_hbm.at[pl.ds(lbase, tm), :],
                          out_sems.at[lslot]).start()

    for i in range(max(0, nblk - nbuf), nblk):
        slot = i % nbuf
        pltpu.make_async_copy(outv.at[slot],
                              out_hbm.at[pl.ds(i * tm, tm), :],
                              out_sems.at[slot]).wait()


def kernel(adj, feature, weight, bias):
    n, d = feature.shape
    f = weight.shape[1]
    tm, nbuf = _TM, _NBUF
    hbm = pl.BlockSpec(memory_space=pltpu.MemorySpace.HBM)
    return pl.pallas_call(
        _gcn_body,
        in_specs=[hbm, hbm, hbm, hbm],
        out_specs=pl.BlockSpec(memory_space=pltpu.MemorySpace.HBM),
        out_shape=jax.ShapeDtypeStruct((n, f), jnp.float32),
        scratch_shapes=[
            pltpu.VMEM((nbuf, tm, n), jnp.float32),
            pltpu.VMEM((n, d), jnp.float32),
            pltpu.VMEM((d, f), jnp.float32),
            pltpu.VMEM((n, f), jnp.float32),
            pltpu.VMEM((n, f), jnp.float32),
            pltpu.VMEM((nbuf, tm, f), jnp.float32),
            pltpu.SemaphoreType.DMA((nbuf,)),
            pltpu.SemaphoreType.DMA,
            pltpu.SemaphoreType.DMA((nbuf,)),
        ],
    )(feature, weight, adj, bias)
